# baseline (speedup 1.0000x reference)
import functools
import jax
import jax.numpy as jnp
from jax import lax
from jax.experimental import pallas as pl
from jax.experimental.pallas import tpu as pltpu
from jax.experimental.pallas import tpu_sc as plsc

_INPUT = 2048
_OUT = 1024
_NBITS = 6
_BATCH = 512
_NLUT = 1 << _NBITS
_NW = 32
_BPW = _BATCH // _NW
_L = 16
_NCHUNK = _OUT // _L
_UPT = _OUT // 16
_MAPW = _NBITS * _OUT


def _lut_body(x_hbm, map_hbm, luts_hbm, out_hbm,
              x_v, map_v, myluts_v, stage_v, bits_v, out_v, shared_bits,
              sem_a, sem_b):
    sid = lax.axis_index("s")
    cid = lax.axis_index("c")
    wid = sid * 2 + cid
    base = wid * _BPW

    luts_cp = pltpu.async_copy(
        luts_hbm.at[pl.ds(sid * _UPT * _NLUT, _UPT * _NLUT)],
        myluts_v, sem_a,
    )
    x_cps = [
        pltpu.async_copy(
            x_hbm.at[base + b], x_v.at[pl.ds(b * _INPUT, _INPUT)], sem_b
        )
        for b in range(_BPW)
    ]
    map_cp = pltpu.async_copy(map_hbm, map_v, sem_b)

    jiota = lax.iota(jnp.int32, _L)

    luts_cp.wait()
    for c in range(_UPT // _L):
        lo = jnp.zeros((_L,), jnp.int32)
        hi = jnp.zeros((_L,), jnp.int32)
        for a in range(_NLUT):
            g = plsc.load_gather(myluts_v, [(c * _L + jiota) * _NLUT + a])
            v = 1 << (a % 32)
            bit = jnp.where(
                g > 0, jnp.int32(v - (1 << 32) if v >= (1 << 31) else v),
                jnp.int32(0),
            )
            if a < 32:
                lo = lo | bit
            else:
                hi = hi | bit
        stage_v[pl.ds(c * _L, _L)] = lo
        stage_v[pl.ds(_UPT + c * _L, _L)] = hi
    pltpu.sync_copy(stage_v.at[pl.ds(0, _UPT)],
                    shared_bits.at[pl.ds(sid * _UPT, _UPT)])
    pltpu.sync_copy(stage_v.at[pl.ds(_UPT, _UPT)],
                    shared_bits.at[pl.ds(_OUT + sid * _UPT, _UPT)])
    plsc.subcore_barrier()
    pltpu.sync_copy(shared_bits, bits_v)

    for cp in x_cps:
        cp.wait()
    map_cp.wait()

    jiota6 = jiota * _NBITS

    @plsc.parallel_loop(0, _NCHUNK, unroll=2)
    def jc_body(jc):
        jb = jc * _L
        lo_w = bits_v[pl.ds(jb, _L)]
        hi_w = bits_v[pl.ds(_OUT + jb, _L)]
        mb = jiota6 + jb * _NBITS
        idxs = [plsc.load_gather(map_v, [mb + k]) for k in range(_NBITS)]
        for b in range(_BPW):
            xrow = x_v.at[pl.ds(b * _INPUT, _INPUT)]
            addr = jnp.zeros((_L,), jnp.int32)
            for k in range(_NBITS):
                g = plsc.load_gather(xrow, [idxs[k]])
                addr = addr + jnp.where(g > 0.0, jnp.int32(1 << k), jnp.int32(0))
            w = jnp.where(addr > 31, hi_w, lo_w)
            bit = (w >> (addr & 31)) & 1
            out_v[pl.ds(b * _OUT + jb, _L)] = bit.astype(jnp.float32)

    out_cps = [
        pltpu.async_copy(
            out_v.at[pl.ds(b * _OUT, _OUT)], out_hbm.at[base + b], sem_a
        )
        for b in range(_BPW)
    ]
    for cp in out_cps:
        cp.wait()


@jax.jit
def _lut_forward(x, map_flat, luts_flat):
    mesh = plsc.VectorSubcoreMesh(core_axis_name="c", subcore_axis_name="s")
    fn = functools.partial(
        pl.kernel,
        mesh=mesh,
        compiler_params=pltpu.CompilerParams(needs_layout_passes=False),
        out_type=jax.ShapeDtypeStruct((_BATCH, _OUT), jnp.float32),
        scratch_types=[
            pltpu.VMEM((_BPW * _INPUT,), jnp.float32),
            pltpu.VMEM((_MAPW,), jnp.int32),
            pltpu.VMEM((_UPT * _NLUT,), jnp.int32),
            pltpu.VMEM((2 * _UPT,), jnp.int32),
            pltpu.VMEM((2 * _OUT,), jnp.int32),
            pltpu.VMEM((_BPW * _OUT,), jnp.float32),
            pltpu.VMEM_SHARED((2 * _OUT,), jnp.int32),
            pltpu.SemaphoreType.DMA,
            pltpu.SemaphoreType.DMA,
        ],
    )(_lut_body)
    return fn(x, map_flat, luts_flat)


def kernel(x, mapping, luts):
    return _lut_forward(
        x,
        mapping.reshape(-1),
        lax.bitcast_convert_type(luts, jnp.int32).reshape(-1),
    )


# speedup vs baseline: 1.0763x; 1.0763x over previous
import functools
import jax
import jax.numpy as jnp
from jax import lax
from jax.experimental import pallas as pl
from jax.experimental.pallas import tpu as pltpu
from jax.experimental.pallas import tpu_sc as plsc

_INPUT = 2048
_OUT = 1024
_NBITS = 6
_BATCH = 512
_NLUT = 1 << _NBITS
_NW = 32
_BPW = _BATCH // _NW
_L = 16
_NCHUNK = _OUT // _L
_UPT = _OUT // 16
_MAPW = _NBITS * _OUT


def _lut_body(x_hbm, map_hbm, luts_hbm, out_hbm,
              x_v, map_v, myluts_v, stage_v, bits_v, out_v, shared_bits,
              sem_a, sem_b):
    sid = lax.axis_index("s")
    cid = lax.axis_index("c")
    wid = sid * 2 + cid
    base = wid * _BPW

    luts_cp = pltpu.async_copy(
        luts_hbm.at[pl.ds(sid * _UPT * _NLUT, _UPT * _NLUT)],
        myluts_v, sem_a,
    )
    x_cps = [
        pltpu.async_copy(
            x_hbm.at[base + b], x_v.at[pl.ds(b * _INPUT, _INPUT)], sem_b
        )
        for b in range(_BPW)
    ]
    map_cp = pltpu.async_copy(map_hbm, map_v, sem_b)

    jiota = lax.iota(jnp.int32, _L)

    luts_cp.wait()
    for c in range(_UPT // _L):
        lo = jnp.zeros((_L,), jnp.int32)
        hi = jnp.zeros((_L,), jnp.int32)
        for a in range(_NLUT):
            g = plsc.load_gather(myluts_v, [(c * _L + jiota) * _NLUT + a])
            v = 1 << (a % 32)
            bit = jnp.where(
                g > 0, jnp.int32(v - (1 << 32) if v >= (1 << 31) else v),
                jnp.int32(0),
            )
            if a < 32:
                lo = lo | bit
            else:
                hi = hi | bit
        stage_v[pl.ds(c * _L, _L)] = lo
        stage_v[pl.ds(_UPT + c * _L, _L)] = hi
    pltpu.sync_copy(stage_v.at[pl.ds(0, _UPT)],
                    shared_bits.at[pl.ds(sid * _UPT, _UPT)])
    pltpu.sync_copy(stage_v.at[pl.ds(_UPT, _UPT)],
                    shared_bits.at[pl.ds(_OUT + sid * _UPT, _UPT)])
    plsc.subcore_barrier()
    pltpu.sync_copy(shared_bits, bits_v)

    for cp in x_cps:
        cp.wait()
    map_cp.wait()

    jiota6 = jiota * _NBITS

    @plsc.parallel_loop(0, _NCHUNK)
    def jc_body(jc):
        jb = jc * _L
        lo_w = bits_v[pl.ds(jb, _L)]
        hi_w = bits_v[pl.ds(_OUT + jb, _L)]
        mb = jiota6 + jb * _NBITS
        idxs = [plsc.load_gather(map_v, [mb + k]) for k in range(_NBITS)]
        for b in range(_BPW):
            xrow = x_v.at[pl.ds(b * _INPUT, _INPUT)]
            addr = jnp.zeros((_L,), jnp.int32)
            for k in range(_NBITS):
                g = plsc.load_gather(xrow, [idxs[k]])
                addr = addr + jnp.where(g > 0.0, jnp.int32(1 << k), jnp.int32(0))
            w = jnp.where(addr > 31, hi_w, lo_w)
            bit = (w >> (addr & 31)) & 1
            out_v[pl.ds(b * _OUT + jb, _L)] = bit.astype(jnp.float32)

    out_cps = [
        pltpu.async_copy(
            out_v.at[pl.ds(b * _OUT, _OUT)], out_hbm.at[base + b], sem_a
        )
        for b in range(_BPW)
    ]
    for cp in out_cps:
        cp.wait()


@jax.jit
def _lut_forward(x, map_flat, luts_flat):
    mesh = plsc.VectorSubcoreMesh(core_axis_name="c", subcore_axis_name="s")
    fn = functools.partial(
        pl.kernel,
        mesh=mesh,
        compiler_params=pltpu.CompilerParams(needs_layout_passes=False),
        out_type=jax.ShapeDtypeStruct((_BATCH, _OUT), jnp.float32),
        scratch_types=[
            pltpu.VMEM((_BPW * _INPUT,), jnp.float32),
            pltpu.VMEM((_MAPW,), jnp.int32),
            pltpu.VMEM((_UPT * _NLUT,), jnp.int32),
            pltpu.VMEM((2 * _UPT,), jnp.int32),
            pltpu.VMEM((2 * _OUT,), jnp.int32),
            pltpu.VMEM((_BPW * _OUT,), jnp.float32),
            pltpu.VMEM_SHARED((2 * _OUT,), jnp.int32),
            pltpu.SemaphoreType.DMA,
            pltpu.SemaphoreType.DMA,
        ],
    )(_lut_body)
    return fn(x, map_flat, luts_flat)


def kernel(x, mapping, luts):
    return _lut_forward(
        x,
        mapping.reshape(-1),
        lax.bitcast_convert_type(luts, jnp.int32).reshape(-1),
    )


# speedup vs baseline: 1.0849x; 1.0081x over previous
import functools
import jax
import jax.numpy as jnp
from jax import lax
from jax.experimental import pallas as pl
from jax.experimental.pallas import tpu as pltpu
from jax.experimental.pallas import tpu_sc as plsc

_INPUT = 2048
_OUT = 1024
_NBITS = 6
_BATCH = 512
_NLUT = 1 << _NBITS
_NW = 32
_BPW = _BATCH // _NW
_L = 16
_NCHUNK = _OUT // _L
_UPT = _OUT // 16
_MAPW = _NBITS * _OUT


def _lut_body(x_hbm, aux_hbm, out_hbm,
              x_v, map_v, myluts_v, stage_v, bits_v, out_v, shared_bits,
              sem_a, sem_b):
    sid = lax.axis_index("s")
    cid = lax.axis_index("c")
    wid = sid * 2 + cid
    base = wid * _BPW

    luts_cp = pltpu.async_copy(
        aux_hbm.at[pl.ds(_MAPW + sid * _UPT * _NLUT, _UPT * _NLUT)],
        myluts_v, sem_a,
    )
    x_cps = [
        pltpu.async_copy(
            x_hbm.at[base + b], x_v.at[pl.ds(b * _INPUT, _INPUT)], sem_b
        )
        for b in range(_BPW)
    ]
    map_cp = pltpu.async_copy(aux_hbm.at[pl.ds(0, _MAPW)], map_v, sem_b)

    jiota = lax.iota(jnp.int32, _L)

    luts_cp.wait()
    for c in range(_UPT // _L):
        lo = jnp.zeros((_L,), jnp.int32)
        hi = jnp.zeros((_L,), jnp.int32)
        for a in range(_NLUT):
            g = plsc.load_gather(myluts_v, [(c * _L + jiota) * _NLUT + a])
            v = 1 << (a % 32)
            bit = jnp.where(
                g > 0, jnp.int32(v - (1 << 32) if v >= (1 << 31) else v),
                jnp.int32(0),
            )
            if a < 32:
                lo = lo | bit
            else:
                hi = hi | bit
        stage_v[pl.ds(c * _L, _L)] = lo
        stage_v[pl.ds(_UPT + c * _L, _L)] = hi
    pltpu.sync_copy(stage_v.at[pl.ds(0, _UPT)],
                    shared_bits.at[pl.ds(sid * _UPT, _UPT)])
    pltpu.sync_copy(stage_v.at[pl.ds(_UPT, _UPT)],
                    shared_bits.at[pl.ds(_OUT + sid * _UPT, _UPT)])
    plsc.subcore_barrier()
    pltpu.sync_copy(shared_bits, bits_v)

    for cp in x_cps:
        cp.wait()
    map_cp.wait()

    @plsc.parallel_loop(0, _NCHUNK)
    def jc_body(jc):
        jb = jc * _L
        lo_w = bits_v[pl.ds(jb, _L)]
        hi_w = bits_v[pl.ds(_OUT + jb, _L)]
        idxs = [map_v[pl.ds(k * _OUT + jb, _L)] for k in range(_NBITS)]
        for b in range(_BPW):
            xrow = x_v.at[pl.ds(b * _INPUT, _INPUT)]
            addr = jnp.zeros((_L,), jnp.int32)
            for k in range(_NBITS):
                g = plsc.load_gather(xrow, [idxs[k]])
                addr = addr + jnp.where(g > 0.0, jnp.int32(1 << k), jnp.int32(0))
            w = jnp.where(addr > 31, hi_w, lo_w)
            bit = (w >> (addr & 31)) & 1
            out_v[pl.ds(b * _OUT + jb, _L)] = bit.astype(jnp.float32)

    out_cps = [
        pltpu.async_copy(
            out_v.at[pl.ds(b * _OUT, _OUT)], out_hbm.at[base + b], sem_a
        )
        for b in range(_BPW)
    ]
    for cp in out_cps:
        cp.wait()


@jax.jit
def _lut_forward(x, aux):
    mesh = plsc.VectorSubcoreMesh(core_axis_name="c", subcore_axis_name="s")
    fn = functools.partial(
        pl.kernel,
        mesh=mesh,
        compiler_params=pltpu.CompilerParams(needs_layout_passes=False),
        out_type=jax.ShapeDtypeStruct((_BATCH, _OUT), jnp.float32),
        scratch_types=[
            pltpu.VMEM((_BPW * _INPUT,), jnp.float32),
            pltpu.VMEM((_MAPW,), jnp.int32),
            pltpu.VMEM((_UPT * _NLUT,), jnp.int32),
            pltpu.VMEM((2 * _UPT,), jnp.int32),
            pltpu.VMEM((2 * _OUT,), jnp.int32),
            pltpu.VMEM((_BPW * _OUT,), jnp.float32),
            pltpu.VMEM_SHARED((2 * _OUT,), jnp.int32),
            pltpu.SemaphoreType.DMA,
            pltpu.SemaphoreType.DMA,
        ],
    )(_lut_body)
    return fn(x, aux)


def kernel(x, mapping, luts):
    aux = jnp.concatenate(
        [
            mapping.T.reshape(-1),
            lax.bitcast_convert_type(luts, jnp.int32).reshape(-1),
        ]
    )
    return _lut_forward(x, aux)


# speedup vs baseline: 1.0858x; 1.0008x over previous
import functools
import jax
import jax.numpy as jnp
from jax import lax
from jax.experimental import pallas as pl
from jax.experimental.pallas import tpu as pltpu
from jax.experimental.pallas import tpu_sc as plsc

_INPUT = 2048
_OUT = 1024
_NBITS = 6
_BATCH = 512
_NLUT = 1 << _NBITS
_NW = 32
_BPW = _BATCH // _NW
_L = 16
_NCHUNK = _OUT // _L
_UPT = _OUT // 16
_MAPW = _NBITS * _OUT


def _lut_body(x_hbm, aux_hbm, out_hbm,
              x_v, map_v, myluts_v, stage_v, bits_v, out_v, shared_bits,
              sem_a, sem_b):
    sid = lax.axis_index("s")
    cid = lax.axis_index("c")
    wid = sid * 2 + cid
    base = wid * _BPW

    luts_cp = pltpu.async_copy(
        aux_hbm.at[pl.ds(_MAPW + sid * _UPT * _NLUT, _UPT * _NLUT)],
        myluts_v, sem_a,
    )
    x_cps = [
        pltpu.async_copy(
            x_hbm.at[base + b], x_v.at[pl.ds(b * _INPUT, _INPUT)], sem_b
        )
        for b in range(_BPW)
    ]
    map_cp = pltpu.async_copy(aux_hbm.at[pl.ds(0, _MAPW)], map_v, sem_b)

    jiota = lax.iota(jnp.int32, _L)

    luts_cp.wait()
    for c in range(_UPT // _L):
        lo = jnp.zeros((_L,), jnp.int32)
        hi = jnp.zeros((_L,), jnp.int32)
        for a in range(_NLUT):
            g = plsc.load_gather(myluts_v, [(c * _L + jiota) * _NLUT + a])
            v = 1 << (a % 32)
            bit = jnp.where(
                g > 0, jnp.int32(v - (1 << 32) if v >= (1 << 31) else v),
                jnp.int32(0),
            )
            if a < 32:
                lo = lo | bit
            else:
                hi = hi | bit
        stage_v[pl.ds(c * _L, _L)] = lo
        stage_v[pl.ds(_UPT + c * _L, _L)] = hi
    pltpu.sync_copy(stage_v.at[pl.ds(0, _UPT)],
                    shared_bits.at[pl.ds(sid * _UPT, _UPT)])
    pltpu.sync_copy(stage_v.at[pl.ds(_UPT, _UPT)],
                    shared_bits.at[pl.ds(_OUT + sid * _UPT, _UPT)])
    plsc.subcore_barrier()
    pltpu.sync_copy(shared_bits, bits_v)

    for cp in x_cps:
        cp.wait()
    map_cp.wait()

    @plsc.parallel_loop(0, _NCHUNK)
    def jc_body(jc):
        jb = jc * _L
        lo_w = bits_v[pl.ds(jb, _L)]
        hi_w = bits_v[pl.ds(_OUT + jb, _L)]
        idxs = [map_v[pl.ds(k * _OUT + jb, _L)] for k in range(_NBITS)]
        for b in range(_BPW):
            xrow = x_v.at[pl.ds(b * _INPUT, _INPUT)]
            addr = jnp.zeros((_L,), jnp.int32)
            for k in range(_NBITS):
                g = plsc.load_gather(xrow, [idxs[k]])
                addr = addr + jnp.where(g > 0.0, jnp.int32(1 << k), jnp.int32(0))
            w = jnp.where(addr > 31, hi_w, lo_w)
            bit = (w >> (addr & 31)) & 1
            out_v[pl.ds(b * _OUT + jb, _L)] = bit.astype(jnp.float32)

    out_cps = [
        pltpu.async_copy(
            out_v.at[pl.ds(b * _OUT, _OUT)], out_hbm.at[base + b], sem_a
        )
        for b in range(_BPW)
    ]
    for cp in out_cps:
        cp.wait()


@jax.jit
def _lut_forward(x, aux):
    mesh = plsc.VectorSubcoreMesh(core_axis_name="c", subcore_axis_name="s")
    fn = functools.partial(
        pl.kernel,
        mesh=mesh,
        compiler_params=pltpu.CompilerParams(
            needs_layout_passes=False,
            disable_semaphore_checks=True,
            skip_device_barrier=True,
        ),
        out_type=jax.ShapeDtypeStruct((_BATCH, _OUT), jnp.float32),
        scratch_types=[
            pltpu.VMEM((_BPW * _INPUT,), jnp.float32),
            pltpu.VMEM((_MAPW,), jnp.int32),
            pltpu.VMEM((_UPT * _NLUT,), jnp.int32),
            pltpu.VMEM((2 * _UPT,), jnp.int32),
            pltpu.VMEM((2 * _OUT,), jnp.int32),
            pltpu.VMEM((_BPW * _OUT,), jnp.float32),
            pltpu.VMEM_SHARED((2 * _OUT,), jnp.int32),
            pltpu.SemaphoreType.DMA,
            pltpu.SemaphoreType.DMA,
        ],
    )(_lut_body)
    return fn(x, aux)


def kernel(x, mapping, luts):
    aux = jnp.concatenate(
        [
            mapping.T.reshape(-1),
            lax.bitcast_convert_type(luts, jnp.int32).reshape(-1),
        ]
    )
    return _lut_forward(x, aux)
